# Initial kernel scaffold; baseline (speedup 1.0000x reference)
import jax
import jax.numpy as jnp
from jax.experimental import pallas as pl


def kernel(x, edge_index, W, b):
    raise NotImplementedError("write your pallas kernel here")


# speedup vs baseline: 33.7531x; 33.7531x over previous
import functools

import jax
import jax.numpy as jnp
from jax import lax
from jax.experimental import pallas as pl
from jax.experimental.pallas import tpu as pltpu
from jax.experimental.pallas import tpu_sc as plsc

N_NODES = 10000
D_FEAT = 128
C = 16
NC = 2
NS = 16
NW = NC * NS
N_PAD = 10240
RPS = N_PAD // NS
RPW = N_PAD // NW
E = 320000
CHUNK = 128
CH = (E // NW + CHUNK - 1) // CHUNK
E_PAD = NW * CH * CHUNK

_MESH = plsc.VectorSubcoreMesh(core_axis_name="c", subcore_axis_name="s")


def _worker_id():
    return lax.axis_index("s") * NC + lax.axis_index("c")


_MM_BLK = 2048


def _mm_body(x_ref, w_ref, o_ref):
    o_ref[...] = jnp.dot(x_ref[...], w_ref[...],
                         preferred_element_type=jnp.float32)


def _matmul(x_pad, wt):
    return pl.pallas_call(
        _mm_body,
        grid=(N_PAD // _MM_BLK,),
        in_specs=[
            pl.BlockSpec((_MM_BLK, D_FEAT), lambda i: (i, 0)),
            pl.BlockSpec((D_FEAT, C), lambda i: (0, 0)),
        ],
        out_specs=pl.BlockSpec((_MM_BLK, C), lambda i: (i, 0)),
        out_shape=jax.ShapeDtypeStruct((N_PAD, C), jnp.float32),
    )(x_pad, wt)


def _make_prop(do_gather):
    scratch = [
        pltpu.VMEM_SHARED((N_PAD, C), jnp.float32),
        pltpu.VMEM((CH, CHUNK), jnp.int32),
        pltpu.VMEM((CHUNK, C), jnp.float32),
        pltpu.SemaphoreType.DMA,
    ]
    if do_gather:
        scratch.append(pltpu.VMEM((CH, CHUNK), jnp.int32))

    def body(g_hbm, src_hbm, dst_hbm, z_hbm, out_hbm, S, dstv, rows, sem,
             *maybe_srcv):
        c = lax.axis_index("c")
        s = lax.axis_index("s")
        wid = _worker_id()
        pltpu.sync_copy(dst_hbm.at[wid], dstv)
        if do_gather:
            srcv = maybe_srcv[0]
            pltpu.sync_copy(src_hbm.at[wid], srcv)
        else:
            def fill(i, carry):
                rows[i] = jnp.ones((C,), jnp.float32)
                return carry
            lax.fori_loop(0, CHUNK, fill, 0)
        sl = pl.ds(s * RPS, RPS)

        @pl.when(c == 0)
        def _():
            pltpu.sync_copy(g_hbm.at[sl], S.at[sl])

        @pl.when(c != 0)
        def _():
            pltpu.sync_copy(z_hbm.at[sl], S.at[sl])

        plsc.subcore_barrier()

        def step(j, carry):
            if do_gather:
                pltpu.async_copy(g_hbm.at[srcv.at[j]], rows, sem).wait()
            pltpu.sync_copy(rows, S.at[dstv.at[j]], add=True)
            return carry

        lax.fori_loop(0, CH, step, 0)
        plsc.subcore_barrier()
        pltpu.sync_copy(S.at[sl], out_hbm.at[pl.ds(c * N_PAD + s * RPS, RPS)])

    return pl.kernel(
        body,
        out_type=jax.ShapeDtypeStruct((NC * N_PAD, C), jnp.float32),
        mesh=_MESH,
        scratch_types=scratch,
        compiler_params=pltpu.CompilerParams(use_tc_tiling_on_sc=False),
    )


_prop_gather = _make_prop(True)
_prop_ones = _make_prop(False)


def _scale1_body(parts_ref, y_ref, g_ref, dis_ref):
    deg = parts_ref[:N_PAD, :] + parts_ref[N_PAD:, :]
    dis = lax.rsqrt(deg)
    dis_ref[...] = dis
    g_ref[...] = dis * y_ref[...]


def _scale1(parts, y):
    return pl.pallas_call(
        _scale1_body,
        out_shape=(jax.ShapeDtypeStruct((N_PAD, C), jnp.float32),
                   jax.ShapeDtypeStruct((N_PAD, C), jnp.float32)),
    )(parts, y)


def _scale2_body(parts_ref, dis_ref, g_ref):
    d = dis_ref[...]
    g_ref[...] = d * d * (parts_ref[:N_PAD, :] + parts_ref[N_PAD:, :])


def _scale2(parts, dis):
    return pl.pallas_call(
        _scale2_body,
        out_shape=jax.ShapeDtypeStruct((N_PAD, C), jnp.float32),
    )(parts, dis)


def _scale3_body(parts_ref, dis_ref, b_ref, o_ref):
    s = parts_ref[:N_PAD, :] + parts_ref[N_PAD:, :]
    o_ref[...] = dis_ref[...] * s + b_ref[...]


def _scale3(parts, dis, b):
    return pl.pallas_call(
        _scale3_body,
        out_shape=jax.ShapeDtypeStruct((N_PAD, C), jnp.float32),
    )(parts, dis, b.reshape(1, C))


def kernel(x, edge_index, W, b):
    src = edge_index[0].astype(jnp.int32)
    dst = edge_index[1].astype(jnp.int32)
    pad = E_PAD - E
    dummy = jnp.full((pad,), N_NODES, jnp.int32)
    src3 = jnp.concatenate([src, dummy]).reshape(NW, CH, CHUNK)
    dst3 = jnp.concatenate([dst, dummy]).reshape(NW, CH, CHUNK)

    x_pad = jnp.pad(x, ((0, N_PAD - N_NODES), (0, 0)))
    y = _matmul(x_pad, W.T)

    ones_tab = jnp.ones((N_PAD, C), jnp.float32)
    zeros_tab = jnp.zeros((N_PAD, C), jnp.float32)

    deg_parts = _prop_ones(ones_tab, src3, dst3, zeros_tab)
    g1, dis = _scale1(deg_parts, y)
    p1 = _prop_gather(g1, src3, dst3, zeros_tab)
    g2 = _scale2(p1, dis)
    p2 = _prop_gather(g2, src3, dst3, zeros_tab)
    out_pad = _scale3(p2, dis, b)
    return out_pad[:N_NODES]


# speedup vs baseline: 55.8276x; 1.6540x over previous
import functools

import jax
import jax.numpy as jnp
from jax import lax
from jax.experimental import pallas as pl
from jax.experimental.pallas import tpu as pltpu
from jax.experimental.pallas import tpu_sc as plsc

N_NODES = 10000
D_FEAT = 128
C = 16
NC = 2
NS = 16
NW = NC * NS
N_PAD = 10240
RPS = N_PAD // NS
RPW = N_PAD // NW
E = 320000
CHUNK = 128
K_BUF = 8
CH = 80
NG = CH // K_BUF
E_PAD = NW * CH * CHUNK

_MESH = plsc.VectorSubcoreMesh(core_axis_name="c", subcore_axis_name="s")


def _worker_id():
    return lax.axis_index("s") * NC + lax.axis_index("c")


_MM_BLK = 2048


def _mm_body(x_ref, w_ref, o_ref):
    o_ref[...] = jnp.dot(x_ref[...], w_ref[...],
                         preferred_element_type=jnp.float32)


def _matmul(x_pad, wt):
    return pl.pallas_call(
        _mm_body,
        grid=(N_PAD // _MM_BLK,),
        in_specs=[
            pl.BlockSpec((_MM_BLK, D_FEAT), lambda i: (i, 0)),
            pl.BlockSpec((D_FEAT, C), lambda i: (0, 0)),
        ],
        out_specs=pl.BlockSpec((_MM_BLK, C), lambda i: (i, 0)),
        out_shape=jax.ShapeDtypeStruct((N_PAD, C), jnp.float32),
    )(x_pad, wt)


def _make_prop(do_gather):
    scratch = [
        pltpu.VMEM_SHARED((N_PAD, C), jnp.float32),
        pltpu.VMEM((CH, CHUNK), jnp.int32),
        pltpu.VMEM((K_BUF, CHUNK, C), jnp.float32),
        pltpu.SemaphoreType.DMA((K_BUF,)),
    ]
    if do_gather:
        scratch += [
            pltpu.VMEM_SHARED((N_PAD, C), jnp.float32),
            pltpu.VMEM((CH, CHUNK), jnp.int32),
            pltpu.SemaphoreType.DMA((K_BUF,)),
        ]

    def body(g_hbm, src_hbm, dst_hbm, z_hbm, out_hbm, S, dstv, rows, ssem,
             *rest):
        c = lax.axis_index("c")
        s = lax.axis_index("s")
        wid = _worker_id()
        pltpu.sync_copy(dst_hbm.at[wid], dstv)
        sl = pl.ds(s * RPS, RPS)
        if do_gather:
            G, srcv, gsem = rest
            pltpu.sync_copy(src_hbm.at[wid], srcv)
            pltpu.sync_copy(g_hbm.at[sl], G.at[sl])
        else:
            def fill(i, carry):
                rows[0, i] = jnp.ones((C,), jnp.float32)
                return carry
            lax.fori_loop(0, CHUNK, fill, 0)

        @pl.when(c == 0)
        def _():
            pltpu.sync_copy(g_hbm.at[sl], S.at[sl])

        @pl.when(c != 0)
        def _():
            pltpu.sync_copy(z_hbm.at[sl], S.at[sl])

        plsc.subcore_barrier()

        def group(g, carry):
            j0 = g * K_BUF
            if do_gather:
                gd = [pltpu.async_copy(G.at[srcv.at[j0 + b]], rows.at[b],
                                       gsem.at[b]) for b in range(K_BUF)]
                sd = []
                for b in range(K_BUF):
                    gd[b].wait()
                    sd.append(pltpu.async_copy(rows.at[b],
                                               S.at[dstv.at[j0 + b]],
                                               ssem.at[b], add=True))
            else:
                sd = [pltpu.async_copy(rows.at[0], S.at[dstv.at[j0 + b]],
                                       ssem.at[b], add=True)
                      for b in range(K_BUF)]
            for b in range(K_BUF):
                sd[b].wait()
            return carry

        lax.fori_loop(0, NG, group, 0)
        plsc.subcore_barrier()
        pltpu.sync_copy(S.at[sl], out_hbm.at[pl.ds(c * N_PAD + s * RPS, RPS)])

    return pl.kernel(
        body,
        out_type=jax.ShapeDtypeStruct((NC * N_PAD, C), jnp.float32),
        mesh=_MESH,
        scratch_types=scratch,
        compiler_params=pltpu.CompilerParams(use_tc_tiling_on_sc=False),
    )


_prop_gather = _make_prop(True)
_prop_ones = _make_prop(False)


def _scale1_body(parts_ref, y_ref, g_ref, dis_ref):
    deg = parts_ref[:N_PAD, :] + parts_ref[N_PAD:, :]
    dis = lax.rsqrt(deg)
    dis_ref[...] = dis
    g_ref[...] = dis * y_ref[...]


def _scale1(parts, y):
    return pl.pallas_call(
        _scale1_body,
        out_shape=(jax.ShapeDtypeStruct((N_PAD, C), jnp.float32),
                   jax.ShapeDtypeStruct((N_PAD, C), jnp.float32)),
    )(parts, y)


def _scale2_body(parts_ref, dis_ref, g_ref):
    d = dis_ref[...]
    g_ref[...] = d * d * (parts_ref[:N_PAD, :] + parts_ref[N_PAD:, :])


def _scale2(parts, dis):
    return pl.pallas_call(
        _scale2_body,
        out_shape=jax.ShapeDtypeStruct((N_PAD, C), jnp.float32),
    )(parts, dis)


def _scale3_body(parts_ref, dis_ref, b_ref, o_ref):
    s = parts_ref[:N_PAD, :] + parts_ref[N_PAD:, :]
    o_ref[...] = dis_ref[...] * s + b_ref[...]


def _scale3(parts, dis, b):
    return pl.pallas_call(
        _scale3_body,
        out_shape=jax.ShapeDtypeStruct((N_PAD, C), jnp.float32),
    )(parts, dis, b.reshape(1, C))


def kernel(x, edge_index, W, b):
    src = edge_index[0].astype(jnp.int32)
    dst = edge_index[1].astype(jnp.int32)
    pad = E_PAD - E
    dummy = jnp.full((pad,), N_NODES, jnp.int32)
    src3 = jnp.concatenate([src, dummy]).reshape(NW, CH, CHUNK)
    dst3 = jnp.concatenate([dst, dummy]).reshape(NW, CH, CHUNK)

    x_pad = jnp.pad(x, ((0, N_PAD - N_NODES), (0, 0)))
    y = _matmul(x_pad, W.T)

    ones_tab = jnp.ones((N_PAD, C), jnp.float32)
    zeros_tab = jnp.zeros((N_PAD, C), jnp.float32)

    deg_parts = _prop_ones(ones_tab, src3, dst3, zeros_tab)
    g1, dis = _scale1(deg_parts, y)
    p1 = _prop_gather(g1, src3, dst3, zeros_tab)
    g2 = _scale2(p1, dis)
    p2 = _prop_gather(g2, src3, dst3, zeros_tab)
    out_pad = _scale3(p2, dis, b)
    return out_pad[:N_NODES]


# speedup vs baseline: 59.7401x; 1.0701x over previous
import functools

import jax
import jax.numpy as jnp
from jax import lax
from jax.experimental import pallas as pl
from jax.experimental.pallas import tpu as pltpu
from jax.experimental.pallas import tpu_sc as plsc

N_NODES = 10000
D_FEAT = 128
C = 16
NC = 2
NS = 16
NW = NC * NS
N_PAD = 10240
RPS = N_PAD // NS
RPW = N_PAD // NW
E = 320000
CHUNK = 128
K_BUF = 8
CH = 80
NG = CH // K_BUF
E_PAD = NW * CH * CHUNK

_MESH = plsc.VectorSubcoreMesh(core_axis_name="c", subcore_axis_name="s")


def _worker_id():
    return lax.axis_index("s") * NC + lax.axis_index("c")


_MM_BLK = 2048


def _mm_body(x_ref, w_ref, d0_ref, d1_ref, g_ref, dis_ref, dis2_ref):
    deg = d0_ref[0] + d1_ref[0]
    dis = lax.rsqrt(deg)
    dis_ref[...] = dis
    dis2_ref[...] = 1.0 / deg
    y = jnp.dot(x_ref[...], w_ref[...], preferred_element_type=jnp.float32)
    g_ref[...] = dis * y


def _matmul_scale1(x_pad, wt, dparts):
    d3 = dparts.reshape(NC, N_PAD, C)
    return pl.pallas_call(
        _mm_body,
        grid=(N_PAD // _MM_BLK,),
        in_specs=[
            pl.BlockSpec((_MM_BLK, D_FEAT), lambda i: (i, 0)),
            pl.BlockSpec((D_FEAT, C), lambda i: (0, 0)),
            pl.BlockSpec((1, _MM_BLK, C), lambda i: (0, i, 0)),
            pl.BlockSpec((1, _MM_BLK, C), lambda i: (1, i, 0)),
        ],
        out_specs=[
            pl.BlockSpec((_MM_BLK, C), lambda i: (i, 0)),
            pl.BlockSpec((_MM_BLK, C), lambda i: (i, 0)),
            pl.BlockSpec((_MM_BLK, C), lambda i: (i, 0)),
        ],
        out_shape=[jax.ShapeDtypeStruct((N_PAD, C), jnp.float32)] * 3,
    )(x_pad, wt, d3, d3)


def _make_prop(mode):
    do_gather = mode != "deg"
    scratch = [
        pltpu.VMEM_SHARED((N_PAD, C), jnp.float32),
        pltpu.VMEM((CH, CHUNK), jnp.int32),
        pltpu.VMEM((K_BUF, CHUNK, C), jnp.float32),
        pltpu.SemaphoreType.DMA((K_BUF,)),
    ]
    if do_gather:
        scratch += [
            pltpu.VMEM_SHARED((N_PAD, C), jnp.float32),
            pltpu.VMEM((CH, CHUNK), jnp.int32),
            pltpu.SemaphoreType.DMA((K_BUF,)),
        ]
    if mode == "mid":
        scratch += [pltpu.VMEM((RPS, C), jnp.float32) for _ in range(4)]

    def body(g_hbm, aux_hbm, src_hbm, dst_hbm, z_hbm, out_hbm, S, dstv, rows,
             ssem, *rest):
        c = lax.axis_index("c")
        s = lax.axis_index("s")
        wid = _worker_id()
        pltpu.sync_copy(dst_hbm.at[wid], dstv)
        sl = pl.ds(s * RPS, RPS)
        if do_gather:
            G, srcv, gsem = rest[:3]
            pltpu.sync_copy(src_hbm.at[wid], srcv)
        if mode == "prop":
            pltpu.sync_copy(g_hbm.at[sl], G.at[sl])
        elif mode == "deg":
            def fill(i, carry):
                rows[0, i] = jnp.ones((C,), jnp.float32)
                return carry
            lax.fori_loop(0, CHUNK, fill, 0)
        elif mode == "mid":
            p0v, p1v, d2v, gv = rest[3:]
            pltpu.sync_copy(g_hbm.at[pl.ds(s * RPS, RPS)], p0v)
            pltpu.sync_copy(g_hbm.at[pl.ds(N_PAD + s * RPS, RPS)], p1v)
            pltpu.sync_copy(aux_hbm.at[sl], d2v)

            def srow(i, carry):
                gv[i] = (p0v[i] + p1v[i]) * d2v[i]
                return carry

            lax.fori_loop(0, RPS, srow, 0)
            pltpu.sync_copy(gv, G.at[sl])

        @pl.when(c == 0)
        def _():
            if mode == "mid":
                pltpu.sync_copy(gv, S.at[sl])
            else:
                pltpu.sync_copy(g_hbm.at[sl], S.at[sl])

        @pl.when(c != 0)
        def _():
            pltpu.sync_copy(z_hbm.at[sl], S.at[sl])

        plsc.subcore_barrier()

        def group(g, carry):
            j0 = g * K_BUF
            if do_gather:
                gd = [pltpu.async_copy(G.at[srcv.at[j0 + b]], rows.at[b],
                                       gsem.at[b]) for b in range(K_BUF)]
                sd = []
                for b in range(K_BUF):
                    gd[b].wait()
                    sd.append(pltpu.async_copy(rows.at[b],
                                               S.at[dstv.at[j0 + b]],
                                               ssem.at[b], add=True))
            else:
                sd = [pltpu.async_copy(rows.at[0], S.at[dstv.at[j0 + b]],
                                       ssem.at[b], add=True)
                      for b in range(K_BUF)]
            for b in range(K_BUF):
                sd[b].wait()
            return carry

        lax.fori_loop(0, NG, group, 0)
        plsc.subcore_barrier()
        pltpu.sync_copy(S.at[sl], out_hbm.at[pl.ds(c * N_PAD + s * RPS, RPS)])

    return pl.kernel(
        body,
        out_type=jax.ShapeDtypeStruct((NC * N_PAD, C), jnp.float32),
        mesh=_MESH,
        scratch_types=scratch,
        compiler_params=pltpu.CompilerParams(use_tc_tiling_on_sc=False),
    )


_prop_deg = _make_prop("deg")
_prop_gather = _make_prop("prop")
_prop_mid = _make_prop("mid")


def _scale3_body(parts_ref, dis_ref, b_ref, o_ref):
    s = parts_ref[:N_PAD, :] + parts_ref[N_PAD:, :]
    o_ref[...] = dis_ref[...] * s + b_ref[...]


def _scale3(parts, dis, b):
    return pl.pallas_call(
        _scale3_body,
        out_shape=jax.ShapeDtypeStruct((N_PAD, C), jnp.float32),
    )(parts, dis, b.reshape(1, C))


def kernel(x, edge_index, W, b):
    src = edge_index[0].astype(jnp.int32)
    dst = edge_index[1].astype(jnp.int32)
    pad = E_PAD - E
    dummy = jnp.full((pad,), N_NODES, jnp.int32)
    src3 = jnp.concatenate([src, dummy]).reshape(NW, CH, CHUNK)
    dst3 = jnp.concatenate([dst, dummy]).reshape(NW, CH, CHUNK)

    x_pad = jnp.pad(x, ((0, N_PAD - N_NODES), (0, 0)))

    ones_tab = jnp.ones((N_PAD, C), jnp.float32)
    zeros_tab = jnp.zeros((N_PAD, C), jnp.float32)

    deg_parts = _prop_deg(ones_tab, zeros_tab, src3, dst3, zeros_tab)
    g1, dis, dis2 = _matmul_scale1(x_pad, W.T, deg_parts)
    p1 = _prop_gather(g1, zeros_tab, src3, dst3, zeros_tab)
    p2 = _prop_mid(p1, dis2, src3, dst3, zeros_tab)
    out_pad = _scale3(p2, dis, b)
    return out_pad[:N_NODES]


# speedup vs baseline: 61.1824x; 1.0241x over previous
import functools

import jax
import jax.numpy as jnp
from jax import lax
from jax.experimental import pallas as pl
from jax.experimental.pallas import tpu as pltpu
from jax.experimental.pallas import tpu_sc as plsc

N_NODES = 10000
D_FEAT = 128
C = 16
NC = 2
NS = 16
NW = NC * NS
N_PAD = 10240
RPS = N_PAD // NS
RPW = N_PAD // NW
E = 320000
CHUNK = 128
K_BUF = 8
LAG = 4
CH = 80
E_PAD = NW * CH * CHUNK

_MESH = plsc.VectorSubcoreMesh(core_axis_name="c", subcore_axis_name="s")


def _worker_id():
    return lax.axis_index("s") * NC + lax.axis_index("c")


_MM_BLK = 2048


def _mm_body(x_ref, w_ref, d0_ref, d1_ref, g_ref, dis_ref, dis2_ref):
    deg = d0_ref[0] + d1_ref[0]
    dis = lax.rsqrt(deg)
    dis_ref[...] = dis
    dis2_ref[...] = 1.0 / deg
    y = jnp.dot(x_ref[...], w_ref[...], preferred_element_type=jnp.float32)
    g_ref[...] = dis * y


def _matmul_scale1(x_pad, wt, dparts):
    d3 = dparts.reshape(NC, N_PAD, C)
    return pl.pallas_call(
        _mm_body,
        grid=(N_PAD // _MM_BLK,),
        in_specs=[
            pl.BlockSpec((_MM_BLK, D_FEAT), lambda i: (i, 0)),
            pl.BlockSpec((D_FEAT, C), lambda i: (0, 0)),
            pl.BlockSpec((1, _MM_BLK, C), lambda i: (0, i, 0)),
            pl.BlockSpec((1, _MM_BLK, C), lambda i: (1, i, 0)),
        ],
        out_specs=[
            pl.BlockSpec((_MM_BLK, C), lambda i: (i, 0)),
            pl.BlockSpec((_MM_BLK, C), lambda i: (i, 0)),
            pl.BlockSpec((_MM_BLK, C), lambda i: (i, 0)),
        ],
        out_shape=[jax.ShapeDtypeStruct((N_PAD, C), jnp.float32)] * 3,
    )(x_pad, wt, d3, d3)


def _make_prop(mode):
    do_gather = mode != "deg"
    scratch = [
        pltpu.VMEM_SHARED((N_PAD, C), jnp.float32),
        pltpu.VMEM((CH, CHUNK), jnp.int32),
        pltpu.VMEM((K_BUF, CHUNK, C), jnp.float32),
        pltpu.SemaphoreType.DMA((K_BUF,)),
    ]
    if do_gather:
        scratch += [
            pltpu.VMEM_SHARED((N_PAD, C), jnp.float32),
            pltpu.VMEM((CH, CHUNK), jnp.int32),
            pltpu.SemaphoreType.DMA((K_BUF,)),
        ]
    if mode == "mid":
        scratch += [pltpu.VMEM((RPS, C), jnp.float32) for _ in range(4)]

    def body(g_hbm, aux_hbm, src_hbm, dst_hbm, z_hbm, out_hbm, S, dstv, rows,
             ssem, *rest):
        c = lax.axis_index("c")
        s = lax.axis_index("s")
        wid = _worker_id()
        pltpu.sync_copy(dst_hbm.at[wid], dstv)
        sl = pl.ds(s * RPS, RPS)
        if do_gather:
            G, srcv, gsem = rest[:3]
            pltpu.sync_copy(src_hbm.at[wid], srcv)
        if mode == "prop":
            pltpu.sync_copy(g_hbm.at[sl], G.at[sl])
        elif mode == "deg":
            def fill(i, carry):
                rows[0, i] = jnp.ones((C,), jnp.float32)
                return carry
            lax.fori_loop(0, CHUNK, fill, 0)
        elif mode == "mid":
            p0v, p1v, d2v, gv = rest[3:]
            pltpu.sync_copy(g_hbm.at[pl.ds(s * RPS, RPS)], p0v)
            pltpu.sync_copy(g_hbm.at[pl.ds(N_PAD + s * RPS, RPS)], p1v)
            pltpu.sync_copy(aux_hbm.at[sl], d2v)

            def srow(i, carry):
                gv[i] = (p0v[i] + p1v[i]) * d2v[i]
                return carry

            lax.fori_loop(0, RPS, srow, 0)
            pltpu.sync_copy(gv, G.at[sl])

        @pl.when(c == 0)
        def _():
            if mode == "mid":
                pltpu.sync_copy(gv, S.at[sl])
            else:
                pltpu.sync_copy(g_hbm.at[sl], S.at[sl])

        @pl.when(c != 0)
        def _():
            pltpu.sync_copy(z_hbm.at[sl], S.at[sl])

        plsc.subcore_barrier()

        if do_gather:
            gdesc = [None] * CH
            sdesc = [None] * CH

            def issue_scatter(j):
                gdesc[j].wait()
                sdesc[j] = pltpu.async_copy(
                    rows.at[j % K_BUF], S.at[dstv.at[j]],
                    ssem.at[j % K_BUF], add=True)

            for j in range(CH):
                if j >= K_BUF:
                    sdesc[j - K_BUF].wait()
                gdesc[j] = pltpu.async_copy(G.at[srcv.at[j]],
                                            rows.at[j % K_BUF],
                                            gsem.at[j % K_BUF])
                if j >= LAG:
                    issue_scatter(j - LAG)
            for j in range(CH - LAG, CH):
                issue_scatter(j)
            for j in range(CH - K_BUF, CH):
                sdesc[j].wait()
        else:
            sdesc = [None] * CH
            for j in range(CH):
                if j >= K_BUF:
                    sdesc[j - K_BUF].wait()
                sdesc[j] = pltpu.async_copy(rows.at[0], S.at[dstv.at[j]],
                                            ssem.at[j % K_BUF], add=True)
            for j in range(CH - K_BUF, CH):
                sdesc[j].wait()
        plsc.subcore_barrier()
        pltpu.sync_copy(S.at[sl], out_hbm.at[pl.ds(c * N_PAD + s * RPS, RPS)])

    return pl.kernel(
        body,
        out_type=jax.ShapeDtypeStruct((NC * N_PAD, C), jnp.float32),
        mesh=_MESH,
        scratch_types=scratch,
        compiler_params=pltpu.CompilerParams(use_tc_tiling_on_sc=False),
    )


_prop_deg = _make_prop("deg")
_prop_gather = _make_prop("prop")
_prop_mid = _make_prop("mid")


def _scale3_body(parts_ref, dis_ref, b_ref, o_ref):
    s = parts_ref[:N_PAD, :] + parts_ref[N_PAD:, :]
    o_ref[...] = dis_ref[...] * s + b_ref[...]


def _scale3(parts, dis, b):
    return pl.pallas_call(
        _scale3_body,
        out_shape=jax.ShapeDtypeStruct((N_PAD, C), jnp.float32),
    )(parts, dis, b.reshape(1, C))


def kernel(x, edge_index, W, b):
    src = edge_index[0].astype(jnp.int32)
    dst = edge_index[1].astype(jnp.int32)
    pad = E_PAD - E
    dummy = jnp.full((pad,), N_NODES, jnp.int32)
    src3 = jnp.concatenate([src, dummy]).reshape(NW, CH, CHUNK)
    dst3 = jnp.concatenate([dst, dummy]).reshape(NW, CH, CHUNK)

    x_pad = jnp.pad(x, ((0, N_PAD - N_NODES), (0, 0)))

    ones_tab = jnp.ones((N_PAD, C), jnp.float32)
    zeros_tab = jnp.zeros((N_PAD, C), jnp.float32)

    deg_parts = _prop_deg(ones_tab, zeros_tab, src3, dst3, zeros_tab)
    g1, dis, dis2 = _matmul_scale1(x_pad, W.T, deg_parts)
    p1 = _prop_gather(g1, zeros_tab, src3, dst3, zeros_tab)
    p2 = _prop_mid(p1, dis2, src3, dst3, zeros_tab)
    out_pad = _scale3(p2, dis, b)
    return out_pad[:N_NODES]
